# Initial kernel scaffold; baseline (speedup 1.0000x reference)
import jax
import jax.numpy as jnp
from jax.experimental import pallas as pl


def kernel(species, coords, atomic_charges, types_map, W1, Wc, b1, w2, atomic_energies):
    raise NotImplementedError("write your pallas kernel here")


# speedup vs baseline: 2.5621x; 2.5621x over previous
import functools

import jax
import jax.numpy as jnp
from jax import lax
from jax.experimental import pallas as pl
from jax.experimental.pallas import tpu as pltpu
from jax.experimental.pallas import tpu_sc as plsc

HARTREE = 2625.4996394798254
NC, NS, L = 2, 16, 16
NW = NC * NS
D = 64


def _sc_body(sp_hbm, c_hbm, q_hbm, tm_hbm, w1_hbm, w4_hbm, w2_hbm, ae_hbm,
             out_hbm, sp_v, c_v, q_v, tm_v, w1_v, w4_v, w2_v, ae_v, ev_v,
             chunk, last):
    wid = lax.axis_index("s") * NC + lax.axis_index("c")
    base = wid * chunk

    pltpu.sync_copy(tm_hbm, tm_v)
    pltpu.sync_copy(w1_hbm, w1_v)
    pltpu.sync_copy(w4_hbm, w4_v)
    pltpu.sync_copy(w2_hbm, w2_v)
    pltpu.sync_copy(ae_hbm, ae_v)

    is_last = wid == NW - 1

    @pl.when(jnp.logical_not(is_last))
    def _():
        pltpu.sync_copy(sp_hbm.at[pl.ds(base, chunk)], sp_v)
        pltpu.sync_copy(q_hbm.at[pl.ds(base, chunk)], q_v)
        pltpu.sync_copy(c_hbm.at[pl.ds(base * 3, chunk * 3)], c_v)

    @pl.when(is_last)
    def _():
        pltpu.sync_copy(sp_hbm.at[pl.ds(base, last)], sp_v.at[pl.ds(0, last)])
        pltpu.sync_copy(q_hbm.at[pl.ds(base, last)], q_v.at[pl.ds(0, last)])
        pltpu.sync_copy(c_hbm.at[pl.ds(base * 3, last * 3)],
                        c_v.at[pl.ds(0, last * 3)])

    w17 = [w4_v[pl.ds(c * L, L)] for c in range(4)]
    wc0 = [w4_v[pl.ds(D + c * L, L)] for c in range(4)]
    wc1 = [w4_v[pl.ds(2 * D + c * L, L)] for c in range(4)]
    wc2 = [w4_v[pl.ds(3 * D + c * L, L)] for c in range(4)]
    w2c = [w2_v[pl.ds(c * L, L)] for c in range(4)]

    zero = jnp.zeros((L,), jnp.float32)
    iota3 = lax.iota(jnp.int32, L) * 3

    def body(grp, carry):
        acc0, acc1, acc2, acc3, e_acc = carry
        b16 = grp * L
        sp16 = sp_v[pl.ds(b16, L)]
        t16 = jnp.maximum(plsc.load_gather(tm_v, [sp16]), 0)
        rb16 = t16 * D
        q16 = q_v[pl.ds(b16, L)]
        cbase = grp * (3 * L)
        x16 = plsc.load_gather(c_v, [cbase + iota3])
        y16 = plsc.load_gather(c_v, [cbase + iota3 + 1])
        z16 = plsc.load_gather(c_v, [cbase + iota3 + 2])
        e_acc = e_acc + plsc.load_gather(ae_v, [t16])
        acc = [acc0, acc1, acc2, acc3]
        for j in range(L):
            rb = rb16[j]
            q = q16[j]
            x = x16[j]
            y = y16[j]
            z = z16[j]
            for c in range(4):
                w1r = w1_v[pl.ds(rb + c * L, L)]
                gv = w1r + q * w17[c] + x * wc0[c] + y * wc1[c] + z * wc2[c]
                acc[c] = acc[c] + jnp.maximum(gv, 0.0) * w2c[c]
        return acc[0], acc[1], acc[2], acc[3], e_acc

    ngroups = jnp.where(is_last, last // L, chunk // L)
    acc0, acc1, acc2, acc3, e_acc = lax.fori_loop(
        0, ngroups, body, (zero, zero, zero, zero, zero))

    ev = acc0 + acc1 + acc2 + acc3 + e_acc
    ev_v[...] = ev * jnp.float32(1.0 / HARTREE)
    pltpu.sync_copy(ev_v, out_hbm.at[wid])


def _tc_reduce_body(p_ref, o_ref):
    o_ref[0, 0] = jnp.sum(p_ref[...])


@jax.jit
def kernel(species, coords, atomic_charges, types_map, W1, Wc, b1, w2,
           atomic_energies):
    n = species.shape[1]
    sp = jnp.squeeze(species, 0)
    chunk = ((n + NW - 1) // NW + 15) // 16 * 16
    last = n - (NW - 1) * chunk

    cflat = jnp.squeeze(coords, 0).reshape(-1)
    tmp = jnp.pad(types_map, (0, 128 - types_map.shape[0]))
    w1flat = (W1[:17] + b1).reshape(-1)
    w4flat = jnp.concatenate([W1[17], 0.1 * Wc[0], 0.1 * Wc[1], 0.1 * Wc[2]])
    aep = jnp.pad(atomic_energies, (0, 32 - atomic_energies.shape[0]))

    mesh = plsc.VectorSubcoreMesh(core_axis_name="c", subcore_axis_name="s",
                                  num_cores=NC, num_subcores=NS)
    sc_call = pl.kernel(
        functools.partial(_sc_body, chunk=chunk, last=last),
        out_type=jax.ShapeDtypeStruct((NW, L), jnp.float32),
        mesh=mesh,
        compiler_params=pltpu.CompilerParams(needs_layout_passes=False),
        scratch_types=[
            pltpu.VMEM((chunk,), jnp.int32),
            pltpu.VMEM((chunk * 3,), jnp.float32),
            pltpu.VMEM((chunk,), jnp.float32),
            pltpu.VMEM((128,), jnp.int32),
            pltpu.VMEM((17 * D,), jnp.float32),
            pltpu.VMEM((4 * D,), jnp.float32),
            pltpu.VMEM((D,), jnp.float32),
            pltpu.VMEM((32,), jnp.float32),
            pltpu.VMEM((L,), jnp.float32),
        ],
    )
    partials = sc_call(sp, cflat, atomic_charges, tmp, w1flat, w4flat, w2,
                       aep)

    energy = pl.pallas_call(
        _tc_reduce_body,
        out_shape=jax.ShapeDtypeStruct((1, 1), jnp.float32),
        in_specs=[pl.BlockSpec(memory_space=pltpu.VMEM)],
        out_specs=pl.BlockSpec(memory_space=pltpu.SMEM),
    )(partials)

    return sp, energy.reshape(1)


# speedup vs baseline: 4.4914x; 1.7530x over previous
import functools

import jax
import jax.numpy as jnp
from jax import lax
from jax.experimental import pallas as pl
from jax.experimental.pallas import tpu as pltpu
from jax.experimental.pallas import tpu_sc as plsc

HARTREE = 2625.4996394798254
NC, NS, L = 2, 16, 16
NW = NC * NS
D = 64


def _sc_body(sp_hbm, c_hbm, q_hbm, tm_hbm, w1t_hbm, w17_hbm, wc0_hbm,
             wbt_hbm, ae_hbm, out_hbm, sp_v, c_v, q_v, tm_v, w1t_v, w17_v,
             wc0_v, wbt_v, ae_v, ev_v, chunk, last):
    wid = lax.axis_index("s") * NC + lax.axis_index("c")
    base = wid * chunk

    pltpu.sync_copy(tm_hbm, tm_v)
    pltpu.sync_copy(w1t_hbm, w1t_v)
    pltpu.sync_copy(w17_hbm, w17_v)
    pltpu.sync_copy(wc0_hbm, wc0_v)
    pltpu.sync_copy(wbt_hbm, wbt_v)
    pltpu.sync_copy(ae_hbm, ae_v)

    is_last = wid == NW - 1

    @pl.when(jnp.logical_not(is_last))
    def _():
        pltpu.sync_copy(sp_hbm.at[pl.ds(base, chunk)], sp_v)
        pltpu.sync_copy(q_hbm.at[pl.ds(base, chunk)], q_v)
        pltpu.sync_copy(c_hbm.at[pl.ds(base * 3, chunk * 3)], c_v)

    @pl.when(is_last)
    def _():
        pltpu.sync_copy(sp_hbm.at[pl.ds(base, last)], sp_v.at[pl.ds(0, last)])
        pltpu.sync_copy(q_hbm.at[pl.ds(base, last)], q_v.at[pl.ds(0, last)])
        pltpu.sync_copy(c_hbm.at[pl.ds(base * 3, last * 3)],
                        c_v.at[pl.ds(0, last * 3)])

    w17c = [w17_v[pl.ds(c * L, L)] for c in range(4)]
    wc0c = [wc0_v[pl.ds(c * L, L)] for c in range(4)]

    zero = jnp.zeros((L,), jnp.float32)
    iota3 = lax.iota(jnp.int32, L) * 3

    def body(grp, carry):
        acc0, acc1, acc2, acc3, e_acc = carry
        b16 = grp * L
        sp16 = sp_v[pl.ds(b16, L)]
        t16 = jnp.maximum(plsc.load_gather(tm_v, [sp16]), 0)
        q16 = q_v[pl.ds(b16, L)]
        cbase = grp * (3 * L)
        x16 = plsc.load_gather(c_v, [cbase + iota3])
        y16 = plsc.load_gather(c_v, [cbase + iota3 + 1])
        z16 = plsc.load_gather(c_v, [cbase + iota3 + 2])
        e_acc = e_acc + plsc.load_gather(ae_v, [t16])
        acc = [acc0, acc1, acc2, acc3]
        for d in range(D):
            c, j = d // L, d % L
            w1d = plsc.load_gather(w1t_v.at[pl.ds(d * 32, 32)], [t16])
            wc1b = wbt_v[pl.ds(d * 48, L)]
            wc2b = wbt_v[pl.ds(d * 48 + L, L)]
            w2b = wbt_v[pl.ds(d * 48 + 2 * L, L)]
            gv = (w1d + q16 * w17c[c][j] + x16 * wc0c[c][j] + y16 * wc1b
                  + z16 * wc2b)
            acc[c] = acc[c] + jnp.maximum(gv, 0.0) * w2b
        return acc[0], acc[1], acc[2], acc[3], e_acc

    ngroups = jnp.where(is_last, last // L, chunk // L)
    acc0, acc1, acc2, acc3, e_acc = lax.fori_loop(
        0, ngroups, body, (zero, zero, zero, zero, zero))

    ev = acc0 + acc1 + acc2 + acc3 + e_acc
    ev_v[...] = ev * jnp.float32(1.0 / HARTREE)
    pltpu.sync_copy(ev_v, out_hbm.at[wid])


def _tc_reduce_body(p_ref, o_ref):
    o_ref[0, 0] = jnp.sum(p_ref[...])


@jax.jit
def kernel(species, coords, atomic_charges, types_map, W1, Wc, b1, w2,
           atomic_energies):
    n = species.shape[1]
    sp = jnp.squeeze(species, 0)
    chunk = ((n + NW - 1) // NW + 15) // 16 * 16
    last = n - (NW - 1) * chunk

    cflat = jnp.squeeze(coords, 0).reshape(-1)
    tmp = jnp.pad(types_map, (0, 128 - types_map.shape[0]))
    w1b = W1[:17] + b1
    w1t = jnp.pad(w1b.T, ((0, 0), (0, 32 - 17))).reshape(-1)
    w17 = W1[17]
    wc0 = 0.1 * Wc[0]
    wbt = jnp.broadcast_to(
        jnp.stack([0.1 * Wc[1], 0.1 * Wc[2], w2], axis=1)[:, :, None],
        (D, 3, L)).reshape(-1)
    aep = jnp.pad(atomic_energies, (0, 32 - atomic_energies.shape[0]))

    mesh = plsc.VectorSubcoreMesh(core_axis_name="c", subcore_axis_name="s",
                                  num_cores=NC, num_subcores=NS)
    sc_call = pl.kernel(
        functools.partial(_sc_body, chunk=chunk, last=last),
        out_type=jax.ShapeDtypeStruct((NW, L), jnp.float32),
        mesh=mesh,
        compiler_params=pltpu.CompilerParams(needs_layout_passes=False),
        scratch_types=[
            pltpu.VMEM((chunk,), jnp.int32),
            pltpu.VMEM((chunk * 3,), jnp.float32),
            pltpu.VMEM((chunk,), jnp.float32),
            pltpu.VMEM((128,), jnp.int32),
            pltpu.VMEM((D * 32,), jnp.float32),
            pltpu.VMEM((D,), jnp.float32),
            pltpu.VMEM((D,), jnp.float32),
            pltpu.VMEM((D * 3 * L,), jnp.float32),
            pltpu.VMEM((32,), jnp.float32),
            pltpu.VMEM((L,), jnp.float32),
        ],
    )
    partials = sc_call(sp, cflat, atomic_charges, tmp, w1t, w17, wc0, wbt,
                       aep)

    energy = pl.pallas_call(
        _tc_reduce_body,
        out_shape=jax.ShapeDtypeStruct((1, 1), jnp.float32),
        in_specs=[pl.BlockSpec(memory_space=pltpu.VMEM)],
        out_specs=pl.BlockSpec(memory_space=pltpu.SMEM),
    )(partials)

    return sp, energy.reshape(1)


# speedup vs baseline: 7.9165x; 1.7626x over previous
import functools

import jax
import jax.numpy as jnp
from jax import lax
from jax.experimental import pallas as pl
from jax.experimental.pallas import tpu as pltpu
from jax.experimental.pallas import tpu_sc as plsc

HARTREE = 2625.4996394798254
NC, NS, L = 2, 16, 16
NW = NC * NS
D = 64


def _sc_body(sp_hbm, x_hbm, y_hbm, z_hbm, q_hbm, tm_hbm, w1t_hbm, wbt_hbm,
             w17_hbm, wc0_hbm, ae_hbm, out_hbm, spo_hbm, sp_v, x_v, y_v,
             z_v, q_v, tm_v, w1t_v, wbt_v, w17_v, wc0_v, ae_v, ev_v, sem,
             chunk, last):
    wid = lax.axis_index("s") * NC + lax.axis_index("c")
    base = wid * chunk
    is_last = wid == NW - 1

    cps = [
        pltpu.async_copy(w17_hbm, w17_v, sem),
        pltpu.async_copy(wc0_hbm, wc0_v, sem),
        pltpu.async_copy(tm_hbm, tm_v, sem),
        pltpu.async_copy(w1t_hbm, w1t_v, sem),
        pltpu.async_copy(wbt_hbm, wbt_v, sem),
        pltpu.async_copy(ae_hbm, ae_v, sem),
    ]

    @pl.when(jnp.logical_not(is_last))
    def _():
        cps2 = [
            pltpu.async_copy(sp_hbm.at[pl.ds(base, chunk)], sp_v, sem),
            pltpu.async_copy(x_hbm.at[pl.ds(base, chunk)], x_v, sem),
            pltpu.async_copy(y_hbm.at[pl.ds(base, chunk)], y_v, sem),
            pltpu.async_copy(z_hbm.at[pl.ds(base, chunk)], z_v, sem),
            pltpu.async_copy(q_hbm.at[pl.ds(base, chunk)], q_v, sem),
        ]
        for c in cps2:
            c.wait()
        pltpu.sync_copy(sp_v, spo_hbm.at[pl.ds(base, chunk)])

    @pl.when(is_last)
    def _():
        cps2 = [
            pltpu.async_copy(sp_hbm.at[pl.ds(base, last)],
                             sp_v.at[pl.ds(0, last)], sem),
            pltpu.async_copy(x_hbm.at[pl.ds(base, last)],
                             x_v.at[pl.ds(0, last)], sem),
            pltpu.async_copy(y_hbm.at[pl.ds(base, last)],
                             y_v.at[pl.ds(0, last)], sem),
            pltpu.async_copy(z_hbm.at[pl.ds(base, last)],
                             z_v.at[pl.ds(0, last)], sem),
            pltpu.async_copy(q_hbm.at[pl.ds(base, last)],
                             q_v.at[pl.ds(0, last)], sem),
        ]
        for c in cps2:
            c.wait()
        pltpu.sync_copy(sp_v.at[pl.ds(0, last)],
                        spo_hbm.at[pl.ds(base, last)])

    for c in cps:
        c.wait()

    w17c = [w17_v[pl.ds(c * L, L)] for c in range(4)]
    wc0c = [wc0_v[pl.ds(c * L, L)] for c in range(4)]

    zero = jnp.zeros((L,), jnp.float32)

    def body(grp, carry):
        acc0, acc1, acc2, acc3, e_acc = carry
        b16 = grp * L
        sp16 = sp_v[pl.ds(b16, L)]
        t16 = jnp.maximum(plsc.load_gather(tm_v, [sp16]), 0)
        q16 = q_v[pl.ds(b16, L)]
        x16 = x_v[pl.ds(b16, L)]
        y16 = y_v[pl.ds(b16, L)]
        z16 = z_v[pl.ds(b16, L)]
        e_acc = e_acc + plsc.load_gather(ae_v, [t16])
        gacc = [zero, zero, zero, zero]
        for d in range(D):
            c, j = divmod(d, L)
            w1d = plsc.load_gather(w1t_v.at[pl.ds(d * 32, 32)], [t16])
            wc1b = wbt_v[pl.ds(d * 48, L)]
            wc2b = wbt_v[pl.ds(d * 48 + L, L)]
            w2b = wbt_v[pl.ds(d * 48 + 2 * L, L)]
            gv = (w1d + q16 * w17c[c][j] + x16 * wc0c[c][j] + y16 * wc1b
                  + z16 * wc2b)
            gacc[c] = gacc[c] + jnp.maximum(gv, 0.0) * w2b
        return (acc0 + gacc[0], acc1 + gacc[1], acc2 + gacc[2],
                acc3 + gacc[3], e_acc)

    ngroups = jnp.where(is_last, last // L, chunk // L)
    acc0, acc1, acc2, acc3, e_acc = lax.fori_loop(
        0, ngroups, body, (zero, zero, zero, zero, zero))

    ev = acc0 + acc1 + acc2 + acc3 + e_acc
    ev_v[...] = ev * jnp.float32(1.0 / HARTREE)
    pltpu.sync_copy(ev_v, out_hbm.at[wid])


def _tc_reduce_body(p_ref, o_ref):
    o_ref[0, 0] = jnp.sum(p_ref[...])


@jax.jit
def kernel(species, coords, atomic_charges, types_map, W1, Wc, b1, w2,
           atomic_energies):
    n = species.shape[1]
    chunk = ((n + NW - 1) // NW + 15) // 16 * 16
    last = n - (NW - 1) * chunk

    xs = coords[0, :, 0]
    ys = coords[0, :, 1]
    zs = coords[0, :, 2]
    tmp = jnp.pad(types_map, (0, 128 - types_map.shape[0]))
    w1b = W1[:17] + b1
    w1t = jnp.pad(w1b.T, ((0, 0), (0, 32 - 17))).reshape(-1)
    wbt = jnp.broadcast_to(
        jnp.stack([0.1 * Wc[1], 0.1 * Wc[2], w2], axis=1)[:, :, None],
        (D, 3, L)).reshape(-1)
    w17 = W1[17]
    wc0 = 0.1 * Wc[0]
    aep = jnp.pad(atomic_energies, (0, 32 - atomic_energies.shape[0]))

    mesh = plsc.VectorSubcoreMesh(core_axis_name="c", subcore_axis_name="s",
                                  num_cores=NC, num_subcores=NS)
    sc_call = pl.kernel(
        functools.partial(_sc_body, chunk=chunk, last=last),
        out_type=(jax.ShapeDtypeStruct((NW, L), jnp.float32),
                  jax.ShapeDtypeStruct((n,), jnp.int32)),
        mesh=mesh,
        compiler_params=pltpu.CompilerParams(needs_layout_passes=False),
        scratch_types=[
            pltpu.VMEM((chunk,), jnp.int32),
            pltpu.VMEM((chunk,), jnp.float32),
            pltpu.VMEM((chunk,), jnp.float32),
            pltpu.VMEM((chunk,), jnp.float32),
            pltpu.VMEM((chunk,), jnp.float32),
            pltpu.VMEM((128,), jnp.int32),
            pltpu.VMEM((D * 32,), jnp.float32),
            pltpu.VMEM((D * 3 * L,), jnp.float32),
            pltpu.VMEM((D,), jnp.float32),
            pltpu.VMEM((D,), jnp.float32),
            pltpu.VMEM((32,), jnp.float32),
            pltpu.VMEM((L,), jnp.float32),
            pltpu.SemaphoreType.DMA,
        ],
    )
    partials, sp_out = sc_call(species.reshape(-1), xs, ys, zs,
                               atomic_charges, tmp, w1t, wbt, w17, wc0, aep)

    energy = pl.pallas_call(
        _tc_reduce_body,
        out_shape=jax.ShapeDtypeStruct((1, 1), jnp.float32),
        in_specs=[pl.BlockSpec(memory_space=pltpu.VMEM)],
        out_specs=pl.BlockSpec(memory_space=pltpu.SMEM),
    )(partials)

    return sp_out, energy.reshape(1)


# speedup vs baseline: 8.8312x; 1.1155x over previous
import functools

import jax
import jax.numpy as jnp
from jax import lax
from jax.experimental import pallas as pl
from jax.experimental.pallas import tpu as pltpu
from jax.experimental.pallas import tpu_sc as plsc

HARTREE = 2625.4996394798254
NC, NS, L = 2, 16, 16
NW = NC * NS
D = 64


def _sc_body(sp_hbm, x_hbm, y_hbm, z_hbm, q_hbm, tm_hbm, w1t_hbm, wbt_hbm,
             w17_hbm, wc0_hbm, ae_hbm, out_hbm, spo_hbm, sp_v, x_v, y_v,
             z_v, q_v, tm_v, w1t_v, wbt_v, w17_v, wc0_v, ae_v, ev_v, sem,
             chunk, last):
    wid = lax.axis_index("s") * NC + lax.axis_index("c")
    base = wid * chunk
    is_last = wid == NW - 1

    cps = [
        pltpu.async_copy(w17_hbm, w17_v, sem),
        pltpu.async_copy(wc0_hbm, wc0_v, sem),
        pltpu.async_copy(tm_hbm, tm_v, sem),
        pltpu.async_copy(w1t_hbm, w1t_v, sem),
        pltpu.async_copy(wbt_hbm, wbt_v, sem),
        pltpu.async_copy(ae_hbm, ae_v, sem),
    ]

    @pl.when(jnp.logical_not(is_last))
    def _():
        cps2 = [
            pltpu.async_copy(sp_hbm.at[pl.ds(base, chunk)], sp_v, sem),
            pltpu.async_copy(x_hbm.at[pl.ds(base, chunk)], x_v, sem),
            pltpu.async_copy(y_hbm.at[pl.ds(base, chunk)], y_v, sem),
            pltpu.async_copy(z_hbm.at[pl.ds(base, chunk)], z_v, sem),
            pltpu.async_copy(q_hbm.at[pl.ds(base, chunk)], q_v, sem),
        ]
        for c in cps2:
            c.wait()
        pltpu.sync_copy(sp_v, spo_hbm.at[pl.ds(base, chunk)])

    @pl.when(is_last)
    def _():
        cps2 = [
            pltpu.async_copy(sp_hbm.at[pl.ds(base, last)],
                             sp_v.at[pl.ds(0, last)], sem),
            pltpu.async_copy(x_hbm.at[pl.ds(base, last)],
                             x_v.at[pl.ds(0, last)], sem),
            pltpu.async_copy(y_hbm.at[pl.ds(base, last)],
                             y_v.at[pl.ds(0, last)], sem),
            pltpu.async_copy(z_hbm.at[pl.ds(base, last)],
                             z_v.at[pl.ds(0, last)], sem),
            pltpu.async_copy(q_hbm.at[pl.ds(base, last)],
                             q_v.at[pl.ds(0, last)], sem),
        ]
        for c in cps2:
            c.wait()
        pltpu.sync_copy(sp_v.at[pl.ds(0, last)],
                        spo_hbm.at[pl.ds(base, last)])

    for c in cps:
        c.wait()

    zero = jnp.zeros((L,), jnp.float32)

    def body(grp, carry):
        acc0, acc1, acc2, acc3, e_acc = carry
        b16 = grp * L
        sp16 = sp_v[pl.ds(b16, L)]
        t16 = jnp.maximum(plsc.load_gather(tm_v, [sp16]), 0)
        q16 = q_v[pl.ds(b16, L)]
        x16 = x_v[pl.ds(b16, L)]
        y16 = y_v[pl.ds(b16, L)]
        z16 = z_v[pl.ds(b16, L)]
        e_acc = e_acc + plsc.load_gather(ae_v, [t16])
        gacc = [zero, zero, zero, zero]
        for c in range(4):
            w17cc = w17_v[pl.ds(c * L, L)]
            wc0cc = wc0_v[pl.ds(c * L, L)]
            ga = zero
            for j in range(L):
                d = c * L + j
                w1d = plsc.load_gather(w1t_v.at[pl.ds(d * 32, 32)], [t16])
                wc1b = wbt_v[pl.ds(d * 48, L)]
                wc2b = wbt_v[pl.ds(d * 48 + L, L)]
                w2b = wbt_v[pl.ds(d * 48 + 2 * L, L)]
                gv = (w1d + q16 * w17cc[j] + x16 * wc0cc[j] + y16 * wc1b
                      + z16 * wc2b)
                ga = ga + jnp.maximum(gv, 0.0) * w2b
            gacc[c] = ga
        return (acc0 + gacc[0], acc1 + gacc[1], acc2 + gacc[2],
                acc3 + gacc[3], e_acc)

    ngroups = jnp.where(is_last, last // L, chunk // L)
    acc0, acc1, acc2, acc3, e_acc = lax.fori_loop(
        0, ngroups, body, (zero, zero, zero, zero, zero))

    ev = acc0 + acc1 + acc2 + acc3 + e_acc
    ev_v[...] = ev * jnp.float32(1.0 / HARTREE)
    pltpu.sync_copy(ev_v, out_hbm.at[wid])


def _tc_reduce_body(p_ref, o_ref):
    o_ref[0, 0] = jnp.sum(p_ref[...])


@jax.jit
def kernel(species, coords, atomic_charges, types_map, W1, Wc, b1, w2,
           atomic_energies):
    n = species.shape[1]
    chunk = ((n + NW - 1) // NW + 15) // 16 * 16
    last = n - (NW - 1) * chunk

    xs = coords[0, :, 0]
    ys = coords[0, :, 1]
    zs = coords[0, :, 2]
    tmp = jnp.pad(types_map, (0, 128 - types_map.shape[0]))
    w1b = W1[:17] + b1
    w1t = jnp.pad(w1b.T, ((0, 0), (0, 32 - 17))).reshape(-1)
    wbt = jnp.broadcast_to(
        jnp.stack([0.1 * Wc[1], 0.1 * Wc[2], w2], axis=1)[:, :, None],
        (D, 3, L)).reshape(-1)
    w17 = W1[17]
    wc0 = 0.1 * Wc[0]
    aep = jnp.pad(atomic_energies, (0, 32 - atomic_energies.shape[0]))

    mesh = plsc.VectorSubcoreMesh(core_axis_name="c", subcore_axis_name="s",
                                  num_cores=NC, num_subcores=NS)
    sc_call = pl.kernel(
        functools.partial(_sc_body, chunk=chunk, last=last),
        out_type=(jax.ShapeDtypeStruct((NW, L), jnp.float32),
                  jax.ShapeDtypeStruct((n,), jnp.int32)),
        mesh=mesh,
        compiler_params=pltpu.CompilerParams(needs_layout_passes=False),
        scratch_types=[
            pltpu.VMEM((chunk,), jnp.int32),
            pltpu.VMEM((chunk,), jnp.float32),
            pltpu.VMEM((chunk,), jnp.float32),
            pltpu.VMEM((chunk,), jnp.float32),
            pltpu.VMEM((chunk,), jnp.float32),
            pltpu.VMEM((128,), jnp.int32),
            pltpu.VMEM((D * 32,), jnp.float32),
            pltpu.VMEM((D * 3 * L,), jnp.float32),
            pltpu.VMEM((D,), jnp.float32),
            pltpu.VMEM((D,), jnp.float32),
            pltpu.VMEM((32,), jnp.float32),
            pltpu.VMEM((L,), jnp.float32),
            pltpu.SemaphoreType.DMA,
        ],
    )
    partials, sp_out = sc_call(species.reshape(-1), xs, ys, zs,
                               atomic_charges, tmp, w1t, wbt, w17, wc0, aep)

    energy = pl.pallas_call(
        _tc_reduce_body,
        out_shape=jax.ShapeDtypeStruct((1, 1), jnp.float32),
        in_specs=[pl.BlockSpec(memory_space=pltpu.VMEM)],
        out_specs=pl.BlockSpec(memory_space=pltpu.SMEM),
    )(partials)

    return sp_out, energy.reshape(1)


# speedup vs baseline: 8.8354x; 1.0005x over previous
import functools

import jax
import jax.numpy as jnp
from jax import lax
from jax.experimental import pallas as pl
from jax.experimental.pallas import tpu as pltpu
from jax.experimental.pallas import tpu_sc as plsc

HARTREE = 2625.4996394798254
NC, NS, L = 2, 16, 16
NW = NC * NS
D = 64


def _sc_body(sp_hbm, c_hbm, q_hbm, tm_hbm, w1t_hbm, wbt_hbm,
             w17_hbm, wc0_hbm, ae_hbm, out_hbm, spo_hbm, sp_v, x_v, y_v,
             z_v, q_v, tm_v, w1t_v, wbt_v, w17_v, wc0_v, ae_v, ev_v, sem,
             chunk, last, n):
    wid = lax.axis_index("s") * NC + lax.axis_index("c")
    base = wid * chunk
    is_last = wid == NW - 1

    cps = [
        pltpu.async_copy(w17_hbm, w17_v, sem),
        pltpu.async_copy(wc0_hbm, wc0_v, sem),
        pltpu.async_copy(tm_hbm, tm_v, sem),
        pltpu.async_copy(w1t_hbm, w1t_v, sem),
        pltpu.async_copy(wbt_hbm, wbt_v, sem),
        pltpu.async_copy(ae_hbm, ae_v, sem),
    ]

    @pl.when(jnp.logical_not(is_last))
    def _():
        cps2 = [
            pltpu.async_copy(sp_hbm.at[pl.ds(base, chunk)], sp_v, sem),
            pltpu.async_copy(c_hbm.at[pl.ds(base, chunk)], x_v, sem),
            pltpu.async_copy(c_hbm.at[pl.ds(n + base, chunk)], y_v, sem),
            pltpu.async_copy(c_hbm.at[pl.ds(2 * n + base, chunk)], z_v, sem),
            pltpu.async_copy(q_hbm.at[pl.ds(base, chunk)], q_v, sem),
        ]
        for c in cps2:
            c.wait()
        pltpu.sync_copy(sp_v, spo_hbm.at[pl.ds(base, chunk)])

    @pl.when(is_last)
    def _():
        cps2 = [
            pltpu.async_copy(sp_hbm.at[pl.ds(base, last)],
                             sp_v.at[pl.ds(0, last)], sem),
            pltpu.async_copy(c_hbm.at[pl.ds(base, last)],
                             x_v.at[pl.ds(0, last)], sem),
            pltpu.async_copy(c_hbm.at[pl.ds(n + base, last)],
                             y_v.at[pl.ds(0, last)], sem),
            pltpu.async_copy(c_hbm.at[pl.ds(2 * n + base, last)],
                             z_v.at[pl.ds(0, last)], sem),
            pltpu.async_copy(q_hbm.at[pl.ds(base, last)],
                             q_v.at[pl.ds(0, last)], sem),
        ]
        for c in cps2:
            c.wait()
        pltpu.sync_copy(sp_v.at[pl.ds(0, last)],
                        spo_hbm.at[pl.ds(base, last)])

    for c in cps:
        c.wait()

    zero = jnp.zeros((L,), jnp.float32)

    def body(grp, carry):
        acc0, acc1, acc2, acc3, e_acc = carry
        b16 = grp * L
        sp16 = sp_v[pl.ds(b16, L)]
        t16 = jnp.maximum(plsc.load_gather(tm_v, [sp16]), 0)
        q16 = q_v[pl.ds(b16, L)]
        x16 = x_v[pl.ds(b16, L)]
        y16 = y_v[pl.ds(b16, L)]
        z16 = z_v[pl.ds(b16, L)]
        e_acc = e_acc + plsc.load_gather(ae_v, [t16])
        gacc = [zero, zero, zero, zero]
        for c in range(4):
            w17cc = w17_v[pl.ds(c * L, L)]
            wc0cc = wc0_v[pl.ds(c * L, L)]
            ga = zero
            for j in range(L):
                d = c * L + j
                w1d = plsc.load_gather(w1t_v.at[pl.ds(d * 32, 32)], [t16])
                wc1b = wbt_v[pl.ds(d * 48, L)]
                wc2b = wbt_v[pl.ds(d * 48 + L, L)]
                w2b = wbt_v[pl.ds(d * 48 + 2 * L, L)]
                gv = (w1d + q16 * w17cc[j] + x16 * wc0cc[j] + y16 * wc1b
                      + z16 * wc2b)
                ga = ga + jnp.maximum(gv, 0.0) * w2b
            gacc[c] = ga
        return (acc0 + gacc[0], acc1 + gacc[1], acc2 + gacc[2],
                acc3 + gacc[3], e_acc)

    ngroups = jnp.where(is_last, last // L, chunk // L)
    acc0, acc1, acc2, acc3, e_acc = lax.fori_loop(
        0, ngroups, body, (zero, zero, zero, zero, zero))

    ev = acc0 + acc1 + acc2 + acc3 + e_acc
    ev_v[...] = ev * jnp.float32(1.0 / HARTREE)
    pltpu.sync_copy(ev_v, out_hbm.at[wid])


def _tc_reduce_body(p_ref, o_ref):
    o_ref[0, 0] = jnp.sum(p_ref[...])


@jax.jit
def kernel(species, coords, atomic_charges, types_map, W1, Wc, b1, w2,
           atomic_energies):
    n = species.shape[1]
    chunk = ((n + NW - 1) // NW + 15) // 16 * 16
    last = n - (NW - 1) * chunk

    cpl = jnp.transpose(coords, (0, 2, 1)).reshape(-1)
    tmp = jnp.pad(types_map, (0, 128 - types_map.shape[0]))
    w1b = W1[:17] + b1
    w1t = jnp.pad(w1b.T, ((0, 0), (0, 32 - 17))).reshape(-1)
    wbt = jnp.broadcast_to(
        jnp.stack([0.1 * Wc[1], 0.1 * Wc[2], w2], axis=1)[:, :, None],
        (D, 3, L)).reshape(-1)
    w17 = W1[17]
    wc0 = 0.1 * Wc[0]
    aep = jnp.pad(atomic_energies, (0, 32 - atomic_energies.shape[0]))

    mesh = plsc.VectorSubcoreMesh(core_axis_name="c", subcore_axis_name="s",
                                  num_cores=NC, num_subcores=NS)
    sc_call = pl.kernel(
        functools.partial(_sc_body, chunk=chunk, last=last, n=n),
        out_type=(jax.ShapeDtypeStruct((NW, L), jnp.float32),
                  jax.ShapeDtypeStruct((n,), jnp.int32)),
        mesh=mesh,
        compiler_params=pltpu.CompilerParams(needs_layout_passes=False),
        scratch_types=[
            pltpu.VMEM((chunk,), jnp.int32),
            pltpu.VMEM((chunk,), jnp.float32),
            pltpu.VMEM((chunk,), jnp.float32),
            pltpu.VMEM((chunk,), jnp.float32),
            pltpu.VMEM((chunk,), jnp.float32),
            pltpu.VMEM((128,), jnp.int32),
            pltpu.VMEM((D * 32,), jnp.float32),
            pltpu.VMEM((D * 3 * L,), jnp.float32),
            pltpu.VMEM((D,), jnp.float32),
            pltpu.VMEM((D,), jnp.float32),
            pltpu.VMEM((32,), jnp.float32),
            pltpu.VMEM((L,), jnp.float32),
            pltpu.SemaphoreType.DMA,
        ],
    )
    partials, sp_out = sc_call(species.reshape(-1), cpl,
                               atomic_charges, tmp, w1t, wbt, w17, wc0, aep)

    energy = pl.pallas_call(
        _tc_reduce_body,
        out_shape=jax.ShapeDtypeStruct((1, 1), jnp.float32),
        in_specs=[pl.BlockSpec(memory_space=pltpu.VMEM)],
        out_specs=pl.BlockSpec(memory_space=pltpu.SMEM),
    )(partials)

    return sp_out, energy.reshape(1)
